# baseline (speedup 1.0000x reference)
import functools

import jax
import jax.numpy as jnp
from jax import lax
from jax.experimental import pallas as pl
from jax.experimental.pallas import tpu as pltpu
from jax.experimental.pallas import tpu_sc as plsc

EMB_DIM = 128
EPS = 1e-5

_NC = 2
_NS = 16
_NW = _NC * _NS

_CH = 128
_SLABS = 5
_ROWS = 4096


def _sc_gather(idx_flat, table):
    n = idx_flat.shape[0]
    b_per_w = n // _NW
    nch = b_per_w // _CH
    mesh = plsc.VectorSubcoreMesh(core_axis_name="c", subcore_axis_name="s")

    @functools.partial(
        pl.kernel,
        out_type=jax.ShapeDtypeStruct((n, EMB_DIM), jnp.float32),
        mesh=mesh,
        scratch_types=[
            pltpu.VMEM((b_per_w,), jnp.int32),
            pltpu.VMEM((_CH, EMB_DIM), jnp.float32),
            pltpu.VMEM((_CH, EMB_DIM), jnp.float32),
            pltpu.SemaphoreType.DMA,
            pltpu.SemaphoreType.DMA,
        ],
    )
    def gather_kernel(idx_hbm, table_hbm, out_hbm, idx_v, buf0, buf1, sem0,
                      sem1):
        wid = lax.axis_index("s") * _NC + lax.axis_index("c")
        base = wid * b_per_w
        pltpu.sync_copy(idx_hbm.at[pl.ds(base, b_per_w)], idx_v)

        def gather_into(g, buf, sem):
            off = pl.multiple_of(g * _CH, _CH)
            pltpu.async_copy(table_hbm.at[idx_v.at[pl.ds(off, _CH)]], buf, sem)

        def scatter_out(g, buf):
            off = pl.multiple_of(g * _CH, _CH)
            pltpu.sync_copy(buf, out_hbm.at[pl.ds(base + off, _CH)])

        gather_into(0, buf0, sem0)

        def step(gg, carry):
            g = pl.multiple_of(gg * 2, 2)
            gather_into(g + 1, buf1, sem1)
            pltpu.make_async_copy(table_hbm.at[pl.ds(0, _CH)], buf0,
                                  sem0).wait()
            scatter_out(g, buf0)

            @pl.when(gg < nch // 2 - 1)
            def _():
                gather_into(g + 2, buf0, sem0)

            pltpu.make_async_copy(table_hbm.at[pl.ds(0, _CH)], buf1,
                                  sem1).wait()
            scatter_out(g + 1, buf1)
            return carry

        lax.fori_loop(0, nch // 2, step, 0)

    return gather_kernel(idx_flat, table)


def _ln_body(big_ref, emb_ref, types_ref, ttab_ref, gamma_ref, beta_ref,
             out_ref):
    del big_ref
    x = emb_ref[...]
    t = types_ref[0, 0, :][:, None]
    te = jnp.where(
        t == 0,
        ttab_ref[0:1, :],
        jnp.where(t == 1, ttab_ref[1:2, :], ttab_ref[2:3, :]),
    )
    y = x + te
    j = jnp.full((EMB_DIM, EMB_DIM), 1.0 / EMB_DIM, dtype=jnp.float32)
    mean = lax.dot(y, j, precision=lax.Precision.DEFAULT)
    sqmean = lax.dot(y * y, j, precision=lax.Precision.DEFAULT)
    var = sqmean - mean * mean
    normed = (y - mean) * lax.rsqrt(var + EPS)
    out_ref[...] = normed * gamma_ref[...] + beta_ref[...]


def _tc_ln_slab(big, emb_s, types3d_s, ttab, gamma2d, beta2d, slab, n):
    n_s = emb_s.shape[0]
    grid = n_s // _ROWS
    blk0 = slab * grid
    return pl.pallas_call(
        _ln_body,
        out_shape=jax.ShapeDtypeStruct((n, EMB_DIM), jnp.float32),
        grid=(grid,),
        in_specs=[
            pl.BlockSpec((8, EMB_DIM), lambda i: (0, 0)),
            pl.BlockSpec((_ROWS, EMB_DIM), lambda i: (i, 0)),
            pl.BlockSpec((1, 1, _ROWS), lambda i: (i, 0, 0)),
            pl.BlockSpec((8, EMB_DIM), lambda i: (0, 0)),
            pl.BlockSpec((1, EMB_DIM), lambda i: (0, 0)),
            pl.BlockSpec((1, EMB_DIM), lambda i: (0, 0)),
        ],
        out_specs=pl.BlockSpec((_ROWS, EMB_DIM),
                               lambda i: (i + blk0, 0)),
        input_output_aliases={0: 0},
    )(big, emb_s, types3d_s, ttab, gamma2d, beta2d)


def kernel(input_tensor, token_type_tensor, token_table, token_type_table,
           gamma, beta):
    batch, seq = input_tensor.shape
    n = batch * seq
    n_s = n // _SLABS
    idx_flat = input_tensor.reshape(n).astype(jnp.int32)
    types_flat = token_type_tensor.reshape(n).astype(jnp.int32)
    ttab = jnp.pad(token_type_table, ((0, 5), (0, 0)))
    gamma2d = gamma.reshape(1, EMB_DIM)
    beta2d = beta.reshape(1, EMB_DIM)

    embs = [_sc_gather(lax.slice(idx_flat, (s * n_s,), ((s + 1) * n_s,)),
                       token_table)
            for s in range(_SLABS)]
    big = jnp.zeros((n, EMB_DIM), dtype=jnp.float32)
    for s in range(_SLABS):
        types3d_s = lax.slice(types_flat, (s * n_s,), ((s + 1) * n_s,)).reshape(
            n_s // _ROWS, 1, _ROWS)
        big = _tc_ln_slab(big, embs[s], types3d_s, ttab, gamma2d, beta2d, s, n)
    return big.reshape(batch, seq, EMB_DIM)


# speedup vs baseline: 1.2784x; 1.2784x over previous
import functools

import jax
import jax.numpy as jnp
from jax import lax
from jax.experimental import pallas as pl
from jax.experimental.pallas import tpu as pltpu
from jax.experimental.pallas import tpu_sc as plsc

EMB_DIM = 128
EPS = 1e-5

_NC = 2
_NS = 16
_NW = _NC * _NS

_CH = 128


def _sc_gather(idx_flat, table):
    n = idx_flat.shape[0]
    b_per_w = n // _NW
    nch = b_per_w // _CH
    mesh = plsc.VectorSubcoreMesh(core_axis_name="c", subcore_axis_name="s")

    @functools.partial(
        pl.kernel,
        out_type=jax.ShapeDtypeStruct((n, EMB_DIM), jnp.float32),
        mesh=mesh,
        scratch_types=[
            pltpu.VMEM((b_per_w,), jnp.int32),
            pltpu.VMEM((_CH, EMB_DIM), jnp.float32),
            pltpu.VMEM((_CH, EMB_DIM), jnp.float32),
            pltpu.SemaphoreType.DMA,
            pltpu.SemaphoreType.DMA,
        ],
    )
    def gather_kernel(idx_hbm, table_hbm, out_hbm, idx_v, buf0, buf1, sem0,
                      sem1):
        wid = lax.axis_index("s") * _NC + lax.axis_index("c")
        base = wid * b_per_w
        pltpu.sync_copy(idx_hbm.at[pl.ds(base, b_per_w)], idx_v)

        def gather_into(g, buf, sem):
            off = pl.multiple_of(g * _CH, _CH)
            pltpu.async_copy(table_hbm.at[idx_v.at[pl.ds(off, _CH)]], buf, sem)

        def scatter_out(g, buf):
            off = pl.multiple_of(g * _CH, _CH)
            pltpu.sync_copy(buf, out_hbm.at[pl.ds(base + off, _CH)])

        gather_into(0, buf0, sem0)

        def step(gg, carry):
            g = pl.multiple_of(gg * 2, 2)
            gather_into(g + 1, buf1, sem1)
            pltpu.make_async_copy(table_hbm.at[pl.ds(0, _CH)], buf0,
                                  sem0).wait()
            scatter_out(g, buf0)

            @pl.when(gg < nch // 2 - 1)
            def _():
                gather_into(g + 2, buf0, sem0)

            pltpu.make_async_copy(table_hbm.at[pl.ds(0, _CH)], buf1,
                                  sem1).wait()
            scatter_out(g + 1, buf1)
            return carry

        lax.fori_loop(0, nch // 2, step, 0)

    return gather_kernel(idx_flat, table)


_ROWS = 8192


def _ln_body(emb_ref, types_ref, ttab_ref, gamma_ref, beta_ref, out_ref):
    x = emb_ref[...]
    t = types_ref[0, 0, :][:, None]
    te = jnp.where(
        t == 0,
        ttab_ref[0:1, :],
        jnp.where(t == 1, ttab_ref[1:2, :], ttab_ref[2:3, :]),
    )
    y = x + te
    j = jnp.full((EMB_DIM, EMB_DIM), 1.0 / EMB_DIM, dtype=jnp.float32)
    mean = lax.dot(y, j, precision=lax.Precision.DEFAULT)
    sqmean = lax.dot(y * y, j, precision=lax.Precision.DEFAULT)
    var = sqmean - mean * mean
    normed = (y - mean) * lax.rsqrt(var + EPS)
    out_ref[...] = normed * gamma_ref[...] + beta_ref[...]


def _tc_ln(emb, types3d, ttab, gamma2d, beta2d):
    n = emb.shape[0]
    grid = n // _ROWS
    return pl.pallas_call(
        _ln_body,
        out_shape=jax.ShapeDtypeStruct((n, EMB_DIM), jnp.float32),
        grid=(grid,),
        in_specs=[
            pl.BlockSpec((_ROWS, EMB_DIM), lambda i: (i, 0)),
            pl.BlockSpec((1, 1, _ROWS), lambda i: (i, 0, 0)),
            pl.BlockSpec((8, EMB_DIM), lambda i: (0, 0)),
            pl.BlockSpec((1, EMB_DIM), lambda i: (0, 0)),
            pl.BlockSpec((1, EMB_DIM), lambda i: (0, 0)),
        ],
        out_specs=pl.BlockSpec((_ROWS, EMB_DIM), lambda i: (i, 0)),
    )(emb, types3d, ttab, gamma2d, beta2d)


def kernel(input_tensor, token_type_tensor, token_table, token_type_table,
           gamma, beta):
    batch, seq = input_tensor.shape
    n = batch * seq
    idx_flat = input_tensor.reshape(n).astype(jnp.int32)
    emb_tok = _sc_gather(idx_flat, token_table)
    types3d = token_type_tensor.reshape(n // _ROWS, 1, _ROWS).astype(jnp.int32)
    ttab = jnp.pad(token_type_table, ((0, 5), (0, 0)))
    out = _tc_ln(emb_tok, types3d, ttab,
                 gamma.reshape(1, EMB_DIM), beta.reshape(1, EMB_DIM))
    return out.reshape(batch, seq, EMB_DIM)


# speedup vs baseline: 1.2927x; 1.0112x over previous
import functools

import jax
import jax.numpy as jnp
from jax import lax
from jax.experimental import pallas as pl
from jax.experimental.pallas import tpu as pltpu
from jax.experimental.pallas import tpu_sc as plsc

EMB_DIM = 128
EPS = 1e-5

_NC = 2
_NS = 16
_NW = _NC * _NS

_CH = 128


def _sc_gather(idx_flat, table):
    n = idx_flat.shape[0]
    b_per_w = n // _NW
    nch = b_per_w // _CH
    mesh = plsc.VectorSubcoreMesh(core_axis_name="c", subcore_axis_name="s")

    nbuf = 4
    assert nch % nbuf == 2

    @functools.partial(
        pl.kernel,
        out_type=jax.ShapeDtypeStruct((n, EMB_DIM), jnp.float32),
        mesh=mesh,
        scratch_types=[
            pltpu.VMEM((b_per_w,), jnp.int32),
            pltpu.VMEM((nbuf, _CH, EMB_DIM), jnp.float32),
        ] + [pltpu.SemaphoreType.DMA] * nbuf,
    )
    def gather_kernel(idx_hbm, table_hbm, out_hbm, idx_v, bufs, *sems):
        wid = lax.axis_index("s") * _NC + lax.axis_index("c")
        base = wid * b_per_w
        pltpu.sync_copy(idx_hbm.at[pl.ds(base, b_per_w)], idx_v)

        def gather_into(g, b):
            off = pl.multiple_of(g * _CH, _CH)
            pltpu.async_copy(table_hbm.at[idx_v.at[pl.ds(off, _CH)]],
                             bufs.at[b], sems[b])

        def wait_gather(b):
            pltpu.make_async_copy(table_hbm.at[pl.ds(0, _CH)], bufs.at[b],
                                  sems[b]).wait()

        def scatter_out(g, b):
            off = pl.multiple_of(g * _CH, _CH)
            pltpu.sync_copy(bufs.at[b], out_hbm.at[pl.ds(base + off, _CH)])

        for b in range(nbuf):
            gather_into(b, b)

        def step(q, carry):
            g0 = pl.multiple_of(q * nbuf, nbuf)
            for b in range(nbuf):
                wait_gather(b)
                scatter_out(g0 + b, b)

                @pl.when(g0 + b + nbuf < nch)
                def _():
                    gather_into(g0 + b + nbuf, b)

            return carry

        lax.fori_loop(0, nch // nbuf, step, 0)
        for b in range(nch % nbuf):
            wait_gather(b)
            scatter_out(nch - (nch % nbuf) + b, b)

    return gather_kernel(idx_flat, table)


_ROWS = 8192


def _ln_body(emb_ref, types_ref, ttab_ref, gamma_ref, beta_ref, out_ref):
    x = emb_ref[...]
    t = types_ref[0, 0, :][:, None]
    te = jnp.where(
        t == 0,
        ttab_ref[0:1, :],
        jnp.where(t == 1, ttab_ref[1:2, :], ttab_ref[2:3, :]),
    )
    y = x + te
    j = jnp.full((EMB_DIM, EMB_DIM), 1.0 / EMB_DIM, dtype=jnp.float32)
    mean = lax.dot(y, j, precision=lax.Precision.DEFAULT)
    sqmean = lax.dot(y * y, j, precision=lax.Precision.DEFAULT)
    var = sqmean - mean * mean
    normed = (y - mean) * lax.rsqrt(var + EPS)
    out_ref[...] = normed * gamma_ref[...] + beta_ref[...]


def _tc_ln(emb, types3d, ttab, gamma2d, beta2d):
    n = emb.shape[0]
    grid = n // _ROWS
    return pl.pallas_call(
        _ln_body,
        out_shape=jax.ShapeDtypeStruct((n, EMB_DIM), jnp.float32),
        grid=(grid,),
        in_specs=[
            pl.BlockSpec((_ROWS, EMB_DIM), lambda i: (i, 0)),
            pl.BlockSpec((1, 1, _ROWS), lambda i: (i, 0, 0)),
            pl.BlockSpec((8, EMB_DIM), lambda i: (0, 0)),
            pl.BlockSpec((1, EMB_DIM), lambda i: (0, 0)),
            pl.BlockSpec((1, EMB_DIM), lambda i: (0, 0)),
        ],
        out_specs=pl.BlockSpec((_ROWS, EMB_DIM), lambda i: (i, 0)),
    )(emb, types3d, ttab, gamma2d, beta2d)


def kernel(input_tensor, token_type_tensor, token_table, token_type_table,
           gamma, beta):
    batch, seq = input_tensor.shape
    n = batch * seq
    idx_flat = input_tensor.reshape(n).astype(jnp.int32)
    emb_tok = _sc_gather(idx_flat, token_table)
    types3d = token_type_tensor.reshape(n // _ROWS, 1, _ROWS).astype(jnp.int32)
    ttab = jnp.pad(token_type_table, ((0, 5), (0, 0)))
    out = _tc_ln(emb_tok, types3d, ttab,
                 gamma.reshape(1, EMB_DIM), beta.reshape(1, EMB_DIM))
    return out.reshape(batch, seq, EMB_DIM)


# speedup vs baseline: 1.3108x; 1.0140x over previous
import functools

import jax
import jax.numpy as jnp
from jax import lax
from jax.experimental import pallas as pl
from jax.experimental.pallas import tpu as pltpu
from jax.experimental.pallas import tpu_sc as plsc

EMB_DIM = 128
EPS = 1e-5

_NC = 2
_NS = 16
_NW = _NC * _NS

_CH = 128


def _sc_gather(idx_flat, table):
    n = idx_flat.shape[0]
    b_per_w = n // _NW
    nch = b_per_w // _CH
    mesh = plsc.VectorSubcoreMesh(core_axis_name="c", subcore_axis_name="s")

    nbuf = 4
    assert nch % nbuf in (0, 2)

    @functools.partial(
        pl.kernel,
        out_type=jax.ShapeDtypeStruct((n, EMB_DIM), jnp.float32),
        mesh=mesh,
        scratch_types=[
            pltpu.VMEM((b_per_w,), jnp.int32),
            pltpu.VMEM((nbuf, _CH, EMB_DIM), jnp.float32),
        ] + [pltpu.SemaphoreType.DMA] * nbuf,
    )
    def gather_kernel(idx_hbm, table_hbm, out_hbm, idx_v, bufs, *sems):
        wid = lax.axis_index("s") * _NC + lax.axis_index("c")
        base = wid * b_per_w
        pltpu.sync_copy(idx_hbm.at[pl.ds(base, b_per_w)], idx_v)

        def gather_into(g, b):
            off = pl.multiple_of(g * _CH, _CH)
            pltpu.async_copy(table_hbm.at[idx_v.at[pl.ds(off, _CH)]],
                             bufs.at[b], sems[b])

        def wait_gather(b):
            pltpu.make_async_copy(table_hbm.at[pl.ds(0, _CH)], bufs.at[b],
                                  sems[b]).wait()

        def scatter_out(g, b):
            off = pl.multiple_of(g * _CH, _CH)
            pltpu.sync_copy(bufs.at[b], out_hbm.at[pl.ds(base + off, _CH)])

        for b in range(nbuf):
            gather_into(b, b)

        def step(q, carry):
            g0 = pl.multiple_of(q * nbuf, nbuf)
            for b in range(nbuf):
                wait_gather(b)
                scatter_out(g0 + b, b)

                @pl.when(g0 + b + nbuf < nch)
                def _():
                    gather_into(g0 + b + nbuf, b)

            return carry

        lax.fori_loop(0, nch // nbuf, step, 0)
        for b in range(nch % nbuf):
            wait_gather(b)
            scatter_out(nch - (nch % nbuf) + b, b)

    return gather_kernel(idx_flat, table)


_ROWS = 8192


def _ln_body(emb_ref, types_ref, ttab_ref, gamma_ref, beta_ref, out_ref):
    _ln_math(emb_ref, types_ref, ttab_ref, gamma_ref, beta_ref, out_ref)


def _ln_body_aliased(big_ref, emb_ref, types_ref, ttab_ref, gamma_ref,
                     beta_ref, out_ref):
    del big_ref
    _ln_math(emb_ref, types_ref, ttab_ref, gamma_ref, beta_ref, out_ref)


def _ln_math(emb_ref, types_ref, ttab_ref, gamma_ref, beta_ref, out_ref):
    x = emb_ref[...]
    t = types_ref[0, 0, :][:, None]
    te = jnp.where(
        t == 0,
        ttab_ref[0:1, :],
        jnp.where(t == 1, ttab_ref[1:2, :], ttab_ref[2:3, :]),
    )
    y = x + te
    j = jnp.full((EMB_DIM, EMB_DIM), 1.0 / EMB_DIM, dtype=jnp.float32)
    mean = lax.dot(y, j, precision=lax.Precision.DEFAULT)
    sqmean = lax.dot(y * y, j, precision=lax.Precision.DEFAULT)
    var = sqmean - mean * mean
    normed = (y - mean) * lax.rsqrt(var + EPS)
    out_ref[...] = normed * gamma_ref[...] + beta_ref[...]


def _tc_ln_slab(big, emb_s, types3d_s, ttab, gamma2d, beta2d, blk0, n):
    n_s = emb_s.shape[0]
    grid = n_s // _ROWS
    common_specs = [
        pl.BlockSpec((_ROWS, EMB_DIM), lambda i: (i, 0)),
        pl.BlockSpec((1, 1, _ROWS), lambda i: (i, 0, 0)),
        pl.BlockSpec((8, EMB_DIM), lambda i: (0, 0)),
        pl.BlockSpec((1, EMB_DIM), lambda i: (0, 0)),
        pl.BlockSpec((1, EMB_DIM), lambda i: (0, 0)),
    ]
    out_spec = pl.BlockSpec((_ROWS, EMB_DIM), lambda i: (i + blk0, 0))
    if big is None:
        return pl.pallas_call(
            _ln_body,
            out_shape=jax.ShapeDtypeStruct((n, EMB_DIM), jnp.float32),
            grid=(grid,),
            in_specs=common_specs,
            out_specs=out_spec,
        )(emb_s, types3d_s, ttab, gamma2d, beta2d)
    return pl.pallas_call(
        _ln_body_aliased,
        out_shape=jax.ShapeDtypeStruct((n, EMB_DIM), jnp.float32),
        grid=(grid,),
        in_specs=[pl.BlockSpec((8, EMB_DIM), lambda i: (0, 0))] + common_specs,
        out_specs=out_spec,
        input_output_aliases={0: 0},
    )(big, emb_s, types3d_s, ttab, gamma2d, beta2d)


_SLAB_UNITS = (13, 12)


def kernel(input_tensor, token_type_tensor, token_table, token_type_table,
           gamma, beta):
    batch, seq = input_tensor.shape
    n = batch * seq
    idx_flat = input_tensor.reshape(n).astype(jnp.int32)
    types_flat = token_type_tensor.reshape(n).astype(jnp.int32)
    ttab = jnp.pad(token_type_table, ((0, 5), (0, 0)))
    gamma2d = gamma.reshape(1, EMB_DIM)
    beta2d = beta.reshape(1, EMB_DIM)

    bounds = []
    row0 = 0
    for u in _SLAB_UNITS:
        bounds.append((row0, row0 + u * _ROWS))
        row0 += u * _ROWS
    assert row0 == n

    embs = [_sc_gather(lax.slice(idx_flat, (lo,), (hi,)), token_table)
            for lo, hi in bounds]
    big = None
    for (lo, hi), emb_s in zip(bounds, embs):
        n_s = hi - lo
        types3d_s = lax.slice(types_flat, (lo,), (hi,)).reshape(
            n_s // _ROWS, 1, _ROWS)
        big = _tc_ln_slab(big, emb_s, types3d_s, ttab, gamma2d, beta2d,
                          lo // _ROWS, n)
    return big.reshape(batch, seq, EMB_DIM)
